# baseline (speedup 1.0000x reference)
import jax
import jax.numpy as jnp
from jax import lax
from jax.experimental import pallas as pl
from jax.experimental.pallas import tpu as pltpu

HIDDEN = 2048
NUM_EXPERTS = 16
TOP_K = 2
TOKENS = 16384
EPS = 1e-05
BLOCK = 256
NEG = -3.0e38


def _router_kernel(x_ref, t_ref, wtp_ref, b_ref, wg_ref, p_ref, i_ref,
                   wtpb_ref):
    @pl.when(pl.program_id(0) == 0)
    def _():
        wtpb_ref[...] = wtp_ref[...].astype(jnp.bfloat16)

    tcb = t_ref[...].astype(jnp.bfloat16)
    mm = lax.dot_general(tcb, wtpb_ref[...], (((1,), (1,)), ((), ())),
                         preferred_element_type=jnp.float32)
    xp = x_ref[...] + mm + b_ref[...]
    logits = lax.dot_general(
        wg_ref[...].astype(jnp.bfloat16), xp.astype(jnp.bfloat16),
        (((1,), (1,)), ((), ())), preferred_element_type=jnp.float32)

    eidx = lax.broadcasted_iota(jnp.int32, (NUM_EXPERTS, BLOCK), 0)
    m1 = jnp.max(logits, axis=0, keepdims=True)
    i1 = jnp.min(jnp.where(logits == m1, eidx, NUM_EXPERTS), axis=0,
                 keepdims=True)
    l2 = jnp.where(eidx == i1, NEG, logits)
    m2 = jnp.max(l2, axis=0, keepdims=True)
    i2 = jnp.min(jnp.where(l2 == m2, eidx, NUM_EXPERTS), axis=0,
                 keepdims=True)
    z = jnp.sum(jnp.exp(logits - m1), axis=0, keepdims=True)
    p1 = 1.0 / z
    p2 = jnp.exp(m2 - m1) / z
    s = p1 + p2 + EPS
    p_ref[...] = jnp.concatenate([p1 / s, p2 / s], axis=0)
    i_ref[...] = jnp.concatenate([i1, i2], axis=0)


def kernel(x, temporal_context, W_tp, b_tp, W_gate):
    grid = TOKENS // BLOCK
    pT, iT = pl.pallas_call(
        _router_kernel,
        grid=(grid,),
        in_specs=[
            pl.BlockSpec((BLOCK, HIDDEN), lambda i: (i, 0)),
            pl.BlockSpec((BLOCK, HIDDEN), lambda i: (i, 0)),
            pl.BlockSpec((HIDDEN, HIDDEN), lambda i: (0, 0)),
            pl.BlockSpec((1, HIDDEN), lambda i: (0, 0)),
            pl.BlockSpec((NUM_EXPERTS, HIDDEN), lambda i: (0, 0)),
        ],
        out_specs=(
            pl.BlockSpec((TOP_K, BLOCK), lambda i: (0, i)),
            pl.BlockSpec((TOP_K, BLOCK), lambda i: (0, i)),
        ),
        out_shape=(
            jax.ShapeDtypeStruct((TOP_K, TOKENS), jnp.float32),
            jax.ShapeDtypeStruct((TOP_K, TOKENS), jnp.int32),
        ),
        scratch_shapes=[pltpu.VMEM((HIDDEN, HIDDEN), jnp.bfloat16)],
    )(x, temporal_context, W_tp, b_tp.reshape(1, HIDDEN), W_gate)
    return pT.T, iT.T


# speedup vs baseline: 1.1001x; 1.1001x over previous
import jax
import jax.numpy as jnp
from jax import lax
from jax.experimental import pallas as pl
from jax.experimental.pallas import tpu as pltpu

HIDDEN = 2048
NUM_EXPERTS = 16
TOP_K = 2
TOKENS = 16384
EPS = 1e-05
BLOCK = 512
NEG = -3.0e38


def _router_kernel(x_ref, t_ref, wtp_ref, b_ref, wg_ref, p_ref, i_ref,
                   wtpb_ref):
    @pl.when(pl.program_id(0) == 0)
    def _():
        wtpb_ref[...] = wtp_ref[...].astype(jnp.bfloat16)

    tcb = t_ref[...].astype(jnp.bfloat16)
    mm = lax.dot_general(tcb, wtpb_ref[...], (((1,), (1,)), ((), ())),
                         preferred_element_type=jnp.float32)
    xp = x_ref[...] + mm + b_ref[...]
    logits = lax.dot_general(
        wg_ref[...].astype(jnp.bfloat16), xp.astype(jnp.bfloat16),
        (((1,), (1,)), ((), ())), preferred_element_type=jnp.float32)

    eidx = lax.broadcasted_iota(jnp.int32, (NUM_EXPERTS, BLOCK), 0)
    m1 = jnp.max(logits, axis=0, keepdims=True)
    i1 = jnp.min(jnp.where(logits == m1, eidx, NUM_EXPERTS), axis=0,
                 keepdims=True)
    l2 = jnp.where(eidx == i1, NEG, logits)
    m2 = jnp.max(l2, axis=0, keepdims=True)
    i2 = jnp.min(jnp.where(l2 == m2, eidx, NUM_EXPERTS), axis=0,
                 keepdims=True)
    z = jnp.sum(jnp.exp(logits - m1), axis=0, keepdims=True)
    p1 = 1.0 / z
    p2 = jnp.exp(m2 - m1) / z
    s = p1 + p2 + EPS
    p_ref[...] = jnp.concatenate([p1 / s, p2 / s], axis=0)
    i_ref[...] = jnp.concatenate([i1, i2], axis=0)


def kernel(x, temporal_context, W_tp, b_tp, W_gate):
    grid = TOKENS // BLOCK
    pT, iT = pl.pallas_call(
        _router_kernel,
        grid=(grid,),
        in_specs=[
            pl.BlockSpec((BLOCK, HIDDEN), lambda i: (i, 0)),
            pl.BlockSpec((BLOCK, HIDDEN), lambda i: (i, 0)),
            pl.BlockSpec((HIDDEN, HIDDEN), lambda i: (0, 0)),
            pl.BlockSpec((1, HIDDEN), lambda i: (0, 0)),
            pl.BlockSpec((NUM_EXPERTS, HIDDEN), lambda i: (0, 0)),
        ],
        out_specs=(
            pl.BlockSpec((TOP_K, BLOCK), lambda i: (0, i)),
            pl.BlockSpec((TOP_K, BLOCK), lambda i: (0, i)),
        ),
        out_shape=(
            jax.ShapeDtypeStruct((TOP_K, TOKENS), jnp.float32),
            jax.ShapeDtypeStruct((TOP_K, TOKENS), jnp.int32),
        ),
        scratch_shapes=[pltpu.VMEM((HIDDEN, HIDDEN), jnp.bfloat16)],
    )(x, temporal_context, W_tp, b_tp.reshape(1, HIDDEN), W_gate)
    return pT.T, iT.T
